# Initial kernel scaffold; baseline (speedup 1.0000x reference)
import jax
import jax.numpy as jnp
from jax.experimental import pallas as pl


def kernel(x, x_edge_index, W1l, b1, W1r, bn_gamma, bn_beta, bn_mean, bn_var, W2l, b2, W2r):
    raise NotImplementedError("write your pallas kernel here")


# speedup vs baseline: 5.0942x; 5.0942x over previous
import functools
import jax
import jax.numpy as jnp
from jax import lax
from jax.experimental import pallas as pl
from jax.experimental.pallas import tpu as pltpu
from jax.experimental.pallas import tpu_sc as plsc

NC = 2
NS = 16
LANES = 16


def _largest_div(n, limit, mult):
    best = mult
    for d in range(mult, limit + 1, mult):
        if n % d == 0:
            best = d
    return best


def _make_sc_agg(N, E, D, with_counts):
    EPT = E // (NC * NS)
    CH = _largest_div(EPT, 128, 8)
    NCHUNK = EPT // CH
    RSTEP = (N // NS) // 8 * 8
    RLEN = N - (NS - 1) * RSTEP
    ZR = _largest_div(RLEN, 128, 8)
    NZ = RLEN // ZR

    mesh = plsc.VectorSubcoreMesh(core_axis_name="c", subcore_axis_name="s",
                                  num_cores=NC, num_subcores=NS)

    out_type = [jax.ShapeDtypeStruct((NC, N, D), jnp.float32)]
    if with_counts:
        out_type.append(jax.ShapeDtypeStruct((NC * N,), jnp.float32))

    scratch = [
        pltpu.VMEM((CH,), jnp.int32),
        pltpu.VMEM((CH,), jnp.int32),
        pltpu.VMEM((CH, D), jnp.float32),
        pltpu.VMEM((ZR, D), jnp.float32),
        pltpu.VMEM_SHARED((N, D), jnp.float32),
        pltpu.SemaphoreType.DMA,
    ]
    if with_counts:
        scratch += [
            pltpu.VMEM((CH,), jnp.float32),
            pltpu.VMEM((RLEN,), jnp.float32),
            pltpu.VMEM_SHARED((N,), jnp.float32),
        ]

    def body(table_hbm, ei_hbm, *refs):
        if with_counts:
            (acc_out, cnt_out, src_v, dst_v, rows_v, z2d, acc_sh, sem,
             ones_v, zrow, cnt_sh) = refs
        else:
            acc_out, src_v, dst_v, rows_v, z2d, acc_sh, sem = refs
        c = lax.axis_index("c")
        s = lax.axis_index("s")

        @pl.loop(0, ZR)
        def _(r):
            for j in range(D // LANES):
                z2d[r, pl.ds(j * LANES, LANES)] = jnp.zeros(
                    (LANES,), jnp.float32)

        row0 = s * RSTEP

        @pl.loop(0, NZ)
        def _(k):
            pltpu.sync_copy(z2d, acc_sh.at[pl.ds(row0 + k * ZR, ZR)])

        if with_counts:
            @pl.loop(0, RLEN // LANES)
            def _(i):
                zrow[pl.ds(i * LANES, LANES)] = jnp.zeros(
                    (LANES,), jnp.float32)

            @pl.loop(0, CH // LANES)
            def _(i):
                ones_v[pl.ds(i * LANES, LANES)] = jnp.ones(
                    (LANES,), jnp.float32)
            pltpu.sync_copy(zrow, cnt_sh.at[pl.ds(row0, RLEN)])

        plsc.subcore_barrier()

        ebase = (c * NS + s) * EPT

        @pl.loop(0, NCHUNK)
        def _(i):
            e0 = ebase + i * CH
            pltpu.sync_copy(ei_hbm.at[pl.ds(e0, CH)], src_v)
            pltpu.sync_copy(ei_hbm.at[pl.ds(E + e0, CH)], dst_v)
            pltpu.async_copy(table_hbm.at[src_v], rows_v, sem).wait()
            pltpu.sync_copy(rows_v, acc_sh.at[dst_v], add=True)
            if with_counts:
                pltpu.sync_copy(ones_v, cnt_sh.at[dst_v], add=True)

        plsc.subcore_barrier()

        @pl.loop(0, NZ)
        def _(k):
            pltpu.sync_copy(acc_sh.at[pl.ds(row0 + k * ZR, ZR)], z2d)
            pltpu.sync_copy(z2d, acc_out.at[c, pl.ds(row0 + k * ZR, ZR)])

        if with_counts:
            pltpu.sync_copy(cnt_sh.at[pl.ds(row0, RLEN)], zrow)
            pltpu.sync_copy(zrow, cnt_out.at[pl.ds(c * N + row0, RLEN)])

    return pl.kernel(body, out_type=tuple(out_type), mesh=mesh,
                     scratch_types=scratch)


def _make_tc_layer(N, D, H, relu):
    R = _largest_div(N, 1024, 8)
    grid = (N // R,)

    def body(agg0, agg1, inv, h, WlT, WrT, sb, bb, out):
        mean = (agg0[...] + agg1[...]) * inv[...]
        t = jnp.dot(mean, WlT[...], preferred_element_type=jnp.float32)
        t = t + jnp.dot(h[...], WrT[...], preferred_element_type=jnp.float32)
        t = t * sb[...] + bb[...]
        if relu:
            t = jnp.maximum(t, 0.0)
        out[...] = t

    row_blk = pl.BlockSpec((R, D), lambda i: (i, 0))
    vec_blk = pl.BlockSpec((R, 1), lambda i: (i, 0))
    w_blk = pl.BlockSpec((D, H), lambda i: (0, 0))
    p_blk = pl.BlockSpec((1, H), lambda i: (0, 0))

    return pl.pallas_call(
        body,
        grid=grid,
        in_specs=[row_blk, row_blk, vec_blk, row_blk, w_blk, w_blk,
                  p_blk, p_blk],
        out_specs=pl.BlockSpec((R, H), lambda i: (i, 0)),
        out_shape=jax.ShapeDtypeStruct((N, H), jnp.float32),
    )


def kernel(x, x_edge_index, W1l, b1, W1r, bn_gamma, bn_beta, bn_mean, bn_var,
           W2l, b2, W2r):
    N, D = x.shape
    E = x_edge_index.shape[1]
    H = W1l.shape[0]
    Z = W2l.shape[0]

    sc_agg1 = _make_sc_agg(N, E, D, with_counts=True)
    sc_agg2 = _make_sc_agg(N, E, H, with_counts=False)
    tc1 = _make_tc_layer(N, D, H, relu=True)
    tc2 = _make_tc_layer(N, H, Z, relu=False)

    ei_flat = x_edge_index.reshape(2 * E)
    acc1, cntp = sc_agg1(x, ei_flat)
    cnt2 = cntp.reshape(NC, N)
    inv = (1.0 / jnp.maximum(cnt2[0] + cnt2[1], 1.0)).reshape(N, 1)

    s1 = bn_gamma / jnp.sqrt(bn_var + 1e-5)
    sb1 = s1.reshape(1, H)
    bb1 = ((b1 - bn_mean) * s1 + bn_beta).reshape(1, H)

    h = tc1(acc1[0], acc1[1], inv, x, W1l.T, W1r.T, sb1, bb1)

    (acc2,) = sc_agg2(h, ei_flat)
    one = jnp.ones((1, Z), jnp.float32)
    z = tc2(acc2[0], acc2[1], inv, h, W2l.T, W2r.T, one, b2.reshape(1, Z))
    return z


# speedup vs baseline: 10.4420x; 2.0498x over previous
import functools
import jax
import jax.numpy as jnp
from jax import lax
from jax.experimental import pallas as pl
from jax.experimental.pallas import tpu as pltpu
from jax.experimental.pallas import tpu_sc as plsc

NC = 2
NS = 16
LANES = 16


def _largest_div(n, limit, mult):
    best = mult
    for d in range(mult, limit + 1, mult):
        if n % d == 0:
            best = d
    return best


def _make_sc_agg(N, E, D, with_counts):
    EPT = E // (NC * NS)
    CH = 128
    NF = EPT // CH
    if NF % 2:
        NF -= 1
    NPAIR = NF // 2
    TAIL = []
    _off = NF * CH
    while _off < EPT:
        _sz = min(CH, EPT - _off)
        TAIL.append((_off, _sz))
        _off += _sz
    TAIL_SIZES = sorted({sz for _, sz in TAIL})
    RSTEP = (N // NS) // 8 * 8
    RLEN = N - (NS - 1) * RSTEP
    ZR = _largest_div(RLEN, 16, 8)
    NZ = RLEN // ZR

    mesh = plsc.VectorSubcoreMesh(core_axis_name="c", subcore_axis_name="s",
                                  num_cores=NC, num_subcores=NS)

    out_type = [jax.ShapeDtypeStruct((NC, N, D), jnp.float32)]
    if with_counts:
        out_type.append(jax.ShapeDtypeStruct((NC * N,), jnp.float32))

    scratch = [
        pltpu.VMEM((EPT,), jnp.int32),
        pltpu.VMEM((CH,), jnp.int32),
        pltpu.VMEM((CH,), jnp.int32),
        pltpu.VMEM((CH, D), jnp.float32),
        pltpu.VMEM((CH, D), jnp.float32),
        pltpu.VMEM((CH,), jnp.int32),
        pltpu.VMEM((CH,), jnp.int32),
        pltpu.VMEM((ZR, D), jnp.float32),
        pltpu.VMEM_SHARED((N, D), jnp.float32),
        pltpu.SemaphoreType.DMA,
        pltpu.SemaphoreType.DMA,
        pltpu.SemaphoreType.DMA,
        pltpu.SemaphoreType.DMA,
        pltpu.SemaphoreType.DMA,
        pltpu.SemaphoreType.DMA,
    ]
    for sz in TAIL_SIZES:
        scratch += [
            pltpu.VMEM((sz, D), jnp.float32),
            pltpu.VMEM((sz,), jnp.int32),
            pltpu.VMEM((sz,), jnp.int32),
        ]
    if with_counts:
        scratch += [
            pltpu.VMEM((CH,), jnp.float32),
            pltpu.VMEM((RLEN,), jnp.float32),
            pltpu.VMEM_SHARED((N,), jnp.float32),
        ]

    def body(table_hbm, ei_hbm, *refs):
        refs = list(refs)
        acc_out = refs.pop(0)
        if with_counts:
            cnt_out = refs.pop(0)
        (dst_all, srcb0, srcb1, rows0, rows1, dstb0, dstb1, z2d, acc_sh,
         semg0, semg1, sems0, sems1, semi0, semi1) = refs[:15]
        refs = refs[15:]
        tail_refs = {}
        for sz in TAIL_SIZES:
            tail_refs[sz] = (refs.pop(0), refs.pop(0), refs.pop(0))
        if with_counts:
            ones_v, zrow, cnt_sh = refs
        c = lax.axis_index("c")
        s = lax.axis_index("s")

        @pl.loop(0, ZR)
        def _(r):
            for j in range(D // LANES):
                z2d[r, pl.ds(j * LANES, LANES)] = jnp.zeros(
                    (LANES,), jnp.float32)

        row0 = s * RSTEP

        @pl.loop(0, NZ)
        def _(k):
            pltpu.sync_copy(z2d, acc_sh.at[pl.ds(row0 + k * ZR, ZR)])

        if with_counts:
            @pl.loop(0, RLEN // LANES)
            def _(i):
                zrow[pl.ds(i * LANES, LANES)] = jnp.zeros(
                    (LANES,), jnp.float32)

            @pl.loop(0, CH // LANES)
            def _(i):
                ones_v[pl.ds(i * LANES, LANES)] = jnp.ones(
                    (LANES,), jnp.float32)
            pltpu.sync_copy(zrow, cnt_sh.at[pl.ds(row0, RLEN)])

        plsc.subcore_barrier()

        ebase = (c * NS + s) * EPT
        pltpu.sync_copy(ei_hbm.at[pl.ds(E + ebase, EPT)], dst_all)

        def start_srcload(i, srcb, semi):
            pltpu.async_copy(ei_hbm.at[pl.ds(ebase + i * CH, CH)], srcb,
                             semi)

        def wait_srcload(srcb, semi):
            pltpu.make_async_copy(ei_hbm.at[pl.ds(0, CH)], srcb,
                                  semi).wait()

        def start_gather(srcb, rows, semg):
            pltpu.async_copy(table_hbm.at[srcb], rows, semg)

        def wait_gather(rows, semg):
            pltpu.make_async_copy(table_hbm.at[pl.ds(0, CH)], rows,
                                  semg).wait()

        def copy_dstb(i, dstb):
            for j in range(CH // LANES):
                dstb[pl.ds(j * LANES, LANES)] = (
                    dst_all[pl.ds(i * CH + j * LANES, LANES)])

        def start_scatter(rows, dstb, sems):
            pltpu.async_copy(rows, acc_sh.at[dstb], sems, add=True)
            if with_counts:
                pltpu.async_copy(ones_v, cnt_sh.at[dstb], sems, add=True)

        def wait_scatter(rows, dstb, sems):
            pltpu.make_async_copy(rows, acc_sh.at[dstb], sems).wait()
            if with_counts:
                pltpu.make_async_copy(ones_v, cnt_sh.at[dstb], sems).wait()

        if NPAIR > 0:
            pltpu.sync_copy(ei_hbm.at[pl.ds(ebase, CH)], srcb0)
            start_gather(srcb0, rows0, semg0)
            start_srcload(1, srcb1, semi1)

            @pl.loop(0, NPAIR)
            def _(k):
                a = k * 2
                wait_gather(rows0, semg0)

                @pl.when(k > 0)
                def _():
                    wait_scatter(rows1, dstb1, sems1)

                wait_srcload(srcb1, semi1)
                start_gather(srcb1, rows1, semg1)

                @pl.when(a + 2 < NF)
                def _():
                    start_srcload(a + 2, srcb0, semi0)

                copy_dstb(a, dstb0)
                start_scatter(rows0, dstb0, sems0)

                wait_gather(rows1, semg1)
                wait_scatter(rows0, dstb0, sems0)

                @pl.when(a + 2 < NF)
                def _():
                    wait_srcload(srcb0, semi0)
                    start_gather(srcb0, rows0, semg0)

                @pl.when(a + 3 < NF)
                def _():
                    start_srcload(a + 3, srcb1, semi1)

                copy_dstb(a + 1, dstb1)
                start_scatter(rows1, dstb1, sems1)

            wait_scatter(rows1, dstb1, sems1)

        for off, sz in TAIL:
            trows, tdst, tsrc = tail_refs[sz]
            pltpu.sync_copy(ei_hbm.at[pl.ds(ebase + off, sz)], tsrc)
            pltpu.async_copy(table_hbm.at[tsrc], trows, semg0)
            for j in range(sz // LANES):
                tdst[pl.ds(j * LANES, LANES)] = (
                    dst_all[pl.ds(off + j * LANES, LANES)])
            pltpu.make_async_copy(table_hbm.at[pl.ds(0, sz)], trows,
                                  semg0).wait()
            pltpu.sync_copy(trows, acc_sh.at[tdst], add=True)
            if with_counts:
                pltpu.sync_copy(ones_v.at[pl.ds(0, sz)], cnt_sh.at[tdst],
                                add=True)

        plsc.subcore_barrier()

        @pl.loop(0, NZ)
        def _(k):
            pltpu.sync_copy(acc_sh.at[pl.ds(row0 + k * ZR, ZR)], z2d)
            pltpu.sync_copy(z2d, acc_out.at[c, pl.ds(row0 + k * ZR, ZR)])

        if with_counts:
            pltpu.sync_copy(cnt_sh.at[pl.ds(row0, RLEN)], zrow)
            pltpu.sync_copy(zrow, cnt_out.at[pl.ds(c * N + row0, RLEN)])

    return pl.kernel(body, out_type=tuple(out_type), mesh=mesh,
                     scratch_types=scratch)


def _make_tc_layer(N, D, H, relu):
    R = _largest_div(N, 1024, 8)
    grid = (N // R,)

    def body(agg0, agg1, inv, h, WlT, WrT, sb, bb, out):
        mean = (agg0[...] + agg1[...]) * inv[...]
        t = jnp.dot(mean, WlT[...], preferred_element_type=jnp.float32)
        t = t + jnp.dot(h[...], WrT[...], preferred_element_type=jnp.float32)
        t = t * sb[...] + bb[...]
        if relu:
            t = jnp.maximum(t, 0.0)
        out[...] = t

    row_blk = pl.BlockSpec((R, D), lambda i: (i, 0))
    vec_blk = pl.BlockSpec((R, 1), lambda i: (i, 0))
    w_blk = pl.BlockSpec((D, H), lambda i: (0, 0))
    p_blk = pl.BlockSpec((1, H), lambda i: (0, 0))

    return pl.pallas_call(
        body,
        grid=grid,
        in_specs=[row_blk, row_blk, vec_blk, row_blk, w_blk, w_blk,
                  p_blk, p_blk],
        out_specs=pl.BlockSpec((R, H), lambda i: (i, 0)),
        out_shape=jax.ShapeDtypeStruct((N, H), jnp.float32),
    )


def kernel(x, x_edge_index, W1l, b1, W1r, bn_gamma, bn_beta, bn_mean, bn_var,
           W2l, b2, W2r):
    N, D = x.shape
    E = x_edge_index.shape[1]
    H = W1l.shape[0]
    Z = W2l.shape[0]

    sc_agg1 = _make_sc_agg(N, E, D, with_counts=True)
    sc_agg2 = _make_sc_agg(N, E, H, with_counts=False)
    tc1 = _make_tc_layer(N, D, H, relu=True)
    tc2 = _make_tc_layer(N, H, Z, relu=False)

    ei_flat = x_edge_index.reshape(2 * E)
    acc1, cntp = sc_agg1(x, ei_flat)
    cnt2 = cntp.reshape(NC, N)
    inv = (1.0 / jnp.maximum(cnt2[0] + cnt2[1], 1.0)).reshape(N, 1)

    s1 = bn_gamma / jnp.sqrt(bn_var + 1e-5)
    sb1 = s1.reshape(1, H)
    bb1 = ((b1 - bn_mean) * s1 + bn_beta).reshape(1, H)

    h = tc1(acc1[0], acc1[1], inv, x, W1l.T, W1r.T, sb1, bb1)

    (acc2,) = sc_agg2(h, ei_flat)
    one = jnp.ones((1, Z), jnp.float32)
    z = tc2(acc2[0], acc2[1], inv, h, W2l.T, W2r.T, one, b2.reshape(1, Z))
    return z


# speedup vs baseline: 10.8405x; 1.0382x over previous
import functools
import jax
import jax.numpy as jnp
from jax import lax
from jax.experimental import pallas as pl
from jax.experimental.pallas import tpu as pltpu
from jax.experimental.pallas import tpu_sc as plsc

NC = 2
NS = 16
LANES = 16


def _largest_div(n, limit, mult):
    best = mult
    for d in range(mult, limit + 1, mult):
        if n % d == 0:
            best = d
    return best


def _make_sc_agg(N, E, D, with_counts):
    EPT = E // (NC * NS)
    CH = 128
    NF = EPT // CH
    if NF % 2:
        NF -= 1
    NPAIR = NF // 2
    TAIL = []
    _off = NF * CH
    while _off < EPT:
        _sz = min(CH, EPT - _off)
        TAIL.append((_off, _sz))
        _off += _sz
    TAIL_SIZES = sorted({sz for _, sz in TAIL})
    RSTEP = (N // NS) // 8 * 8
    RLEN = N - (NS - 1) * RSTEP
    ZR = _largest_div(RLEN, 16, 8)
    NZ = RLEN // ZR

    mesh = plsc.VectorSubcoreMesh(core_axis_name="c", subcore_axis_name="s",
                                  num_cores=NC, num_subcores=NS)

    out_type = [jax.ShapeDtypeStruct((NC, N, D), jnp.float32)]
    if with_counts:
        out_type.append(jax.ShapeDtypeStruct((NC * N,), jnp.float32))

    scratch = [
        pltpu.VMEM((EPT,), jnp.int32),
        pltpu.VMEM((CH,), jnp.int32),
        pltpu.VMEM((CH,), jnp.int32),
        pltpu.VMEM((CH, D), jnp.float32),
        pltpu.VMEM((CH, D), jnp.float32),
        pltpu.VMEM((CH,), jnp.int32),
        pltpu.VMEM((CH,), jnp.int32),
        pltpu.VMEM((ZR, D), jnp.float32),
        pltpu.VMEM_SHARED((N, D), jnp.float32),
        pltpu.SemaphoreType.DMA,
        pltpu.SemaphoreType.DMA,
        pltpu.SemaphoreType.DMA,
        pltpu.SemaphoreType.DMA,
        pltpu.SemaphoreType.DMA,
        pltpu.SemaphoreType.DMA,
    ]
    for sz in TAIL_SIZES:
        scratch += [
            pltpu.VMEM((sz, D), jnp.float32),
            pltpu.VMEM((sz,), jnp.int32),
            pltpu.VMEM((sz,), jnp.int32),
        ]
    if with_counts:
        scratch += [
            pltpu.VMEM((CH,), jnp.float32),
            pltpu.VMEM((RLEN,), jnp.float32),
            pltpu.VMEM_SHARED((N,), jnp.float32),
        ]

    def body(table_hbm, ei_hbm, *refs):
        refs = list(refs)
        acc_out = refs.pop(0)
        if with_counts:
            cnt_out = refs.pop(0)
        (dst_all, srcb0, srcb1, rows0, rows1, dstb0, dstb1, z2d, acc_sh,
         semg0, semg1, sems0, sems1, semi0, semi1) = refs[:15]
        refs = refs[15:]
        tail_refs = {}
        for sz in TAIL_SIZES:
            tail_refs[sz] = (refs.pop(0), refs.pop(0), refs.pop(0))
        if with_counts:
            ones_v, zrow, cnt_sh = refs
        c = lax.axis_index("c")
        s = lax.axis_index("s")
        ebase = (c * NS + s) * EPT

        pltpu.async_copy(ei_hbm.at[pl.ds(E + ebase, EPT)], dst_all, semi0)
        if NPAIR > 0:
            pltpu.async_copy(ei_hbm.at[pl.ds(ebase, CH)], srcb0, semi1)

        @pl.loop(0, ZR)
        def _(r):
            for j in range(D // LANES):
                z2d[r, pl.ds(j * LANES, LANES)] = jnp.zeros(
                    (LANES,), jnp.float32)

        row0 = s * RSTEP

        @pl.loop(0, NZ)
        def _(k):
            pltpu.sync_copy(z2d, acc_sh.at[pl.ds(row0 + k * ZR, ZR)])

        if with_counts:
            @pl.loop(0, RLEN // LANES)
            def _(i):
                zrow[pl.ds(i * LANES, LANES)] = jnp.zeros(
                    (LANES,), jnp.float32)

            @pl.loop(0, CH // LANES)
            def _(i):
                ones_v[pl.ds(i * LANES, LANES)] = jnp.ones(
                    (LANES,), jnp.float32)
            pltpu.sync_copy(zrow, cnt_sh.at[pl.ds(row0, RLEN)])

        def start_srcload(i, srcb, semi):
            pltpu.async_copy(ei_hbm.at[pl.ds(ebase + i * CH, CH)], srcb,
                             semi)

        def wait_srcload(srcb, semi):
            pltpu.make_async_copy(ei_hbm.at[pl.ds(0, CH)], srcb,
                                  semi).wait()

        def start_gather(srcb, rows, semg):
            pltpu.async_copy(table_hbm.at[srcb], rows, semg)

        def wait_gather(rows, semg):
            pltpu.make_async_copy(table_hbm.at[pl.ds(0, CH)], rows,
                                  semg).wait()

        def copy_dstb(i, dstb):
            for j in range(CH // LANES):
                dstb[pl.ds(j * LANES, LANES)] = (
                    dst_all[pl.ds(i * CH + j * LANES, LANES)])

        def start_scatter(rows, dstb, sems):
            pltpu.async_copy(rows, acc_sh.at[dstb], sems, add=True)
            if with_counts:
                pltpu.async_copy(ones_v, cnt_sh.at[dstb], sems, add=True)

        def wait_scatter(rows, dstb, sems):
            pltpu.make_async_copy(rows, acc_sh.at[dstb], sems).wait()
            if with_counts:
                pltpu.make_async_copy(ones_v, cnt_sh.at[dstb], sems).wait()

        if NPAIR > 0:
            pltpu.make_async_copy(ei_hbm.at[pl.ds(0, CH)], srcb0,
                                  semi1).wait()
            start_gather(srcb0, rows0, semg0)
            start_srcload(1, srcb1, semi1)
        pltpu.make_async_copy(ei_hbm.at[pl.ds(0, EPT)], dst_all,
                              semi0).wait()

        plsc.subcore_barrier()

        if NPAIR > 0:
            @pl.loop(0, NPAIR)
            def _(k):
                a = k * 2
                wait_gather(rows0, semg0)

                @pl.when(k > 0)
                def _():
                    wait_scatter(rows1, dstb1, sems1)

                wait_srcload(srcb1, semi1)
                start_gather(srcb1, rows1, semg1)

                @pl.when(a + 2 < NF)
                def _():
                    start_srcload(a + 2, srcb0, semi0)

                copy_dstb(a, dstb0)
                start_scatter(rows0, dstb0, sems0)

                wait_gather(rows1, semg1)
                wait_scatter(rows0, dstb0, sems0)

                @pl.when(a + 2 < NF)
                def _():
                    wait_srcload(srcb0, semi0)
                    start_gather(srcb0, rows0, semg0)

                @pl.when(a + 3 < NF)
                def _():
                    start_srcload(a + 3, srcb1, semi1)

                copy_dstb(a + 1, dstb1)
                start_scatter(rows1, dstb1, sems1)

            wait_scatter(rows1, dstb1, sems1)

        for off, sz in TAIL:
            trows, tdst, tsrc = tail_refs[sz]
            pltpu.sync_copy(ei_hbm.at[pl.ds(ebase + off, sz)], tsrc)
            pltpu.async_copy(table_hbm.at[tsrc], trows, semg0)
            for j in range(sz // LANES):
                tdst[pl.ds(j * LANES, LANES)] = (
                    dst_all[pl.ds(off + j * LANES, LANES)])
            pltpu.make_async_copy(table_hbm.at[pl.ds(0, sz)], trows,
                                  semg0).wait()
            pltpu.sync_copy(trows, acc_sh.at[tdst], add=True)
            if with_counts:
                pltpu.sync_copy(ones_v.at[pl.ds(0, sz)], cnt_sh.at[tdst],
                                add=True)

        plsc.subcore_barrier()

        @pl.loop(0, NZ)
        def _(k):
            pltpu.sync_copy(acc_sh.at[pl.ds(row0 + k * ZR, ZR)], z2d)
            pltpu.sync_copy(z2d, acc_out.at[c, pl.ds(row0 + k * ZR, ZR)])

        if with_counts:
            pltpu.sync_copy(cnt_sh.at[pl.ds(row0, RLEN)], zrow)
            pltpu.sync_copy(zrow, cnt_out.at[pl.ds(c * N + row0, RLEN)])

    return pl.kernel(body, out_type=tuple(out_type), mesh=mesh,
                     scratch_types=scratch)


def _make_tc_layer1(N, D, H):
    R = _largest_div(N, 1024, 8)

    def body(acc, cnt, x, Wl, Wr, sb, bb, out, inv_out):
        cntb = cnt[0] + cnt[1]
        inv = 1.0 / jnp.maximum(cntb, 1.0)
        inv_out[...] = inv
        mean = (acc[0] + acc[1]) * inv
        dn = (((1,), (1,)), ((), ()))
        t = lax.dot_general(mean, Wl[...], dn,
                            preferred_element_type=jnp.float32)
        t = t + lax.dot_general(x[...], Wr[...], dn,
                                preferred_element_type=jnp.float32)
        t = t * sb[...] + bb[...]
        out[...] = jnp.maximum(t, 0.0)

    return pl.pallas_call(
        body,
        grid=(N // R,),
        in_specs=[
            pl.BlockSpec((2, R, D), lambda i: (0, i, 0)),
            pl.BlockSpec((2, R, 1), lambda i: (0, i, 0)),
            pl.BlockSpec((R, D), lambda i: (i, 0)),
            pl.BlockSpec((H, D), lambda i: (0, 0)),
            pl.BlockSpec((H, D), lambda i: (0, 0)),
            pl.BlockSpec((1, H), lambda i: (0, 0)),
            pl.BlockSpec((1, H), lambda i: (0, 0)),
        ],
        out_specs=[
            pl.BlockSpec((R, H), lambda i: (i, 0)),
            pl.BlockSpec((R, 1), lambda i: (i, 0)),
        ],
        out_shape=[
            jax.ShapeDtypeStruct((N, H), jnp.float32),
            jax.ShapeDtypeStruct((N, 1), jnp.float32),
        ],
    )


def _make_tc_layer2(N, H, Z):
    R = _largest_div(N, 1024, 8)

    def body(acc, inv, h, Wl, Wr, bb, out):
        mean = (acc[0] + acc[1]) * inv[...]
        dn = (((1,), (1,)), ((), ()))
        t = lax.dot_general(mean, Wl[...], dn,
                            preferred_element_type=jnp.float32)
        t = t + lax.dot_general(h[...], Wr[...], dn,
                                preferred_element_type=jnp.float32)
        out[...] = t + bb[...]

    return pl.pallas_call(
        body,
        grid=(N // R,),
        in_specs=[
            pl.BlockSpec((2, R, H), lambda i: (0, i, 0)),
            pl.BlockSpec((R, 1), lambda i: (i, 0)),
            pl.BlockSpec((R, H), lambda i: (i, 0)),
            pl.BlockSpec((Z, H), lambda i: (0, 0)),
            pl.BlockSpec((Z, H), lambda i: (0, 0)),
            pl.BlockSpec((1, Z), lambda i: (0, 0)),
        ],
        out_specs=pl.BlockSpec((R, Z), lambda i: (i, 0)),
        out_shape=jax.ShapeDtypeStruct((N, Z), jnp.float32),
    )


def kernel(x, x_edge_index, W1l, b1, W1r, bn_gamma, bn_beta, bn_mean, bn_var,
           W2l, b2, W2r):
    N, D = x.shape
    E = x_edge_index.shape[1]
    H = W1l.shape[0]
    Z = W2l.shape[0]

    sc_agg1 = _make_sc_agg(N, E, D, with_counts=True)
    sc_agg2 = _make_sc_agg(N, E, H, with_counts=False)
    tc1 = _make_tc_layer1(N, D, H)
    tc2 = _make_tc_layer2(N, H, Z)

    ei_flat = x_edge_index.reshape(2 * E)
    acc1, cntp = sc_agg1(x, ei_flat)
    cnt3 = cntp.reshape(NC, N, 1)

    s1 = bn_gamma / jnp.sqrt(bn_var + 1e-5)
    sb1 = s1.reshape(1, H)
    bb1 = ((b1 - bn_mean) * s1 + bn_beta).reshape(1, H)

    h, inv = tc1(acc1, cnt3, x, W1l, W1r, sb1, bb1)

    (acc2,) = sc_agg2(h, ei_flat)
    z = tc2(acc2, inv, h, W2l, W2r, b2.reshape(1, Z))
    return z


# speedup vs baseline: 11.4816x; 1.0591x over previous
import functools
import jax
import jax.numpy as jnp
from jax import lax
from jax.experimental import pallas as pl
from jax.experimental.pallas import tpu as pltpu
from jax.experimental.pallas import tpu_sc as plsc

NC = 2
NS = 16
LANES = 16


def _largest_div(n, limit, mult):
    best = mult
    for d in range(mult, limit + 1, mult):
        if n % d == 0:
            best = d
    return best


def _make_sc_agg(N, E, D, with_counts):
    EPT = E // (NC * NS)
    CH = 128
    NF = EPT // CH
    if NF % 2:
        NF -= 1
    NPAIR = NF // 2
    TAIL = []
    _off = NF * CH
    while _off < EPT:
        _sz = min(CH, EPT - _off)
        TAIL.append((_off, _sz))
        _off += _sz
    TAIL_SIZES = sorted({sz for _, sz in TAIL})
    RSTEP = (N // NS) // 8 * 8
    RLEN = N - (NS - 1) * RSTEP
    ZR = _largest_div(RLEN, 16, 8)
    NZ = RLEN // ZR
    PB = _largest_div(RLEN, CH, 8)
    NPUB = RLEN // PB

    mesh = plsc.VectorSubcoreMesh(core_axis_name="c", subcore_axis_name="s",
                                  num_cores=NC, num_subcores=NS)

    out_type = [jax.ShapeDtypeStruct((NC, N, D), jnp.float32)]
    if with_counts:
        out_type.append(jax.ShapeDtypeStruct((NC * N,), jnp.float32))

    scratch = [
        pltpu.VMEM((EPT,), jnp.int32),
        pltpu.VMEM((CH,), jnp.int32),
        pltpu.VMEM((CH,), jnp.int32),
        pltpu.VMEM((CH, D), jnp.float32),
        pltpu.VMEM((CH, D), jnp.float32),
        pltpu.VMEM((CH,), jnp.int32),
        pltpu.VMEM((CH,), jnp.int32),
        pltpu.VMEM((ZR, D), jnp.float32),
        pltpu.VMEM_SHARED((N, D), jnp.float32),
        pltpu.SemaphoreType.DMA,
        pltpu.SemaphoreType.DMA,
        pltpu.SemaphoreType.DMA,
        pltpu.SemaphoreType.DMA,
        pltpu.SemaphoreType.DMA,
        pltpu.SemaphoreType.DMA,
        pltpu.SemaphoreType.DMA,
    ]
    for sz in TAIL_SIZES:
        scratch += [
            pltpu.VMEM((sz, D), jnp.float32),
            pltpu.VMEM((sz,), jnp.int32),
            pltpu.VMEM((sz,), jnp.int32),
        ]
    if with_counts:
        scratch += [
            pltpu.VMEM((CH,), jnp.float32),
            pltpu.VMEM((RLEN,), jnp.float32),
            pltpu.VMEM_SHARED((N,), jnp.float32),
        ]

    def body(table_hbm, ei_hbm, *refs):
        refs = list(refs)
        acc_out = refs.pop(0)
        if with_counts:
            cnt_out = refs.pop(0)
        (dst_all, srcb0, srcb1, rows0, rows1, dstb0, dstb1, z2d, acc_sh,
         semg0, semg1, sems0, sems1, semi0, semi1, semz) = refs[:16]
        refs = refs[16:]
        tail_refs = {}
        for sz in TAIL_SIZES:
            tail_refs[sz] = (refs.pop(0), refs.pop(0), refs.pop(0))
        if with_counts:
            ones_v, zrow, cnt_sh = refs
        c = lax.axis_index("c")
        s = lax.axis_index("s")
        ebase = (c * NS + s) * EPT

        pltpu.async_copy(ei_hbm.at[pl.ds(E + ebase, EPT)], dst_all, semi0)
        if NPAIR > 0:
            pltpu.async_copy(ei_hbm.at[pl.ds(ebase, CH)], srcb0, semi1)

        @pl.loop(0, ZR)
        def _(r):
            for j in range(D // LANES):
                z2d[r, pl.ds(j * LANES, LANES)] = jnp.zeros(
                    (LANES,), jnp.float32)

        row0 = s * RSTEP

        @pl.loop(0, NZ)
        def _(k):
            pltpu.async_copy(z2d, acc_sh.at[pl.ds(row0 + k * ZR, ZR)],
                             semz)

        @pl.loop(0, NZ)
        def _(k):
            pltpu.make_async_copy(z2d, acc_sh.at[pl.ds(row0 + k * ZR, ZR)],
                                  semz).wait()

        if with_counts:
            @pl.loop(0, RLEN // LANES)
            def _(i):
                zrow[pl.ds(i * LANES, LANES)] = jnp.zeros(
                    (LANES,), jnp.float32)

            @pl.loop(0, CH // LANES)
            def _(i):
                ones_v[pl.ds(i * LANES, LANES)] = jnp.ones(
                    (LANES,), jnp.float32)
            pltpu.sync_copy(zrow, cnt_sh.at[pl.ds(row0, RLEN)])

        def start_srcload(i, srcb, semi):
            pltpu.async_copy(ei_hbm.at[pl.ds(ebase + i * CH, CH)], srcb,
                             semi)

        def wait_srcload(srcb, semi):
            pltpu.make_async_copy(ei_hbm.at[pl.ds(0, CH)], srcb,
                                  semi).wait()

        def start_gather(srcb, rows, semg):
            pltpu.async_copy(table_hbm.at[srcb], rows, semg)

        def wait_gather(rows, semg):
            pltpu.make_async_copy(table_hbm.at[pl.ds(0, CH)], rows,
                                  semg).wait()

        def copy_dstb(i, dstb):
            for j in range(CH // LANES):
                dstb[pl.ds(j * LANES, LANES)] = (
                    dst_all[pl.ds(i * CH + j * LANES, LANES)])

        def start_scatter(rows, dstb, sems):
            pltpu.async_copy(rows, acc_sh.at[dstb], sems, add=True)
            if with_counts:
                pltpu.async_copy(ones_v, cnt_sh.at[dstb], sems, add=True)

        def wait_scatter(rows, dstb, sems):
            pltpu.make_async_copy(rows, acc_sh.at[dstb], sems).wait()
            if with_counts:
                pltpu.make_async_copy(ones_v, cnt_sh.at[dstb], sems).wait()

        if NPAIR > 0:
            pltpu.make_async_copy(ei_hbm.at[pl.ds(0, CH)], srcb0,
                                  semi1).wait()
            start_gather(srcb0, rows0, semg0)
            start_srcload(1, srcb1, semi1)
        pltpu.make_async_copy(ei_hbm.at[pl.ds(0, EPT)], dst_all,
                              semi0).wait()

        plsc.subcore_barrier()

        if NPAIR > 0:
            @pl.loop(0, NPAIR)
            def _(k):
                a = k * 2
                wait_gather(rows0, semg0)

                @pl.when(k > 0)
                def _():
                    wait_scatter(rows1, dstb1, sems1)

                wait_srcload(srcb1, semi1)
                start_gather(srcb1, rows1, semg1)

                @pl.when(a + 2 < NF)
                def _():
                    start_srcload(a + 2, srcb0, semi0)

                copy_dstb(a, dstb0)
                start_scatter(rows0, dstb0, sems0)

                wait_gather(rows1, semg1)
                wait_scatter(rows0, dstb0, sems0)

                @pl.when(a + 2 < NF)
                def _():
                    wait_srcload(srcb0, semi0)
                    start_gather(srcb0, rows0, semg0)

                @pl.when(a + 3 < NF)
                def _():
                    start_srcload(a + 3, srcb1, semi1)

                copy_dstb(a + 1, dstb1)
                start_scatter(rows1, dstb1, sems1)

            wait_scatter(rows1, dstb1, sems1)

        for off, sz in TAIL:
            trows, tdst, tsrc = tail_refs[sz]
            pltpu.sync_copy(ei_hbm.at[pl.ds(ebase + off, sz)], tsrc)
            pltpu.async_copy(table_hbm.at[tsrc], trows, semg0)
            for j in range(sz // LANES):
                tdst[pl.ds(j * LANES, LANES)] = (
                    dst_all[pl.ds(off + j * LANES, LANES)])
            pltpu.make_async_copy(table_hbm.at[pl.ds(0, sz)], trows,
                                  semg0).wait()
            pltpu.sync_copy(trows, acc_sh.at[tdst], add=True)
            if with_counts:
                pltpu.sync_copy(ones_v.at[pl.ds(0, sz)], cnt_sh.at[tdst],
                                add=True)

        plsc.subcore_barrier()

        pub = [(rows0, semg0), (rows1, semg1)]
        for k in range(NPUB):
            buf, sem = pub[k % 2]
            lo = row0 + k * PB
            if k >= 2:
                pltpu.make_async_copy(
                    buf.at[pl.ds(0, PB)],
                    acc_out.at[c, pl.ds(row0 + (k - 2) * PB, PB)],
                    sem).wait()
            pltpu.sync_copy(acc_sh.at[pl.ds(lo, PB)], buf.at[pl.ds(0, PB)])
            pltpu.async_copy(buf.at[pl.ds(0, PB)],
                             acc_out.at[c, pl.ds(lo, PB)], sem)
        for k in range(max(0, NPUB - 2), NPUB):
            buf, sem = pub[k % 2]
            pltpu.make_async_copy(buf.at[pl.ds(0, PB)],
                                  acc_out.at[c, pl.ds(row0 + k * PB, PB)],
                                  sem).wait()

        if with_counts:
            pltpu.sync_copy(cnt_sh.at[pl.ds(row0, RLEN)], zrow)
            pltpu.sync_copy(zrow, cnt_out.at[pl.ds(c * N + row0, RLEN)])

    return pl.kernel(body, out_type=tuple(out_type), mesh=mesh,
                     scratch_types=scratch)


def _make_tc_layer1(N, D, H):
    R = _largest_div(N, 1024, 8)

    def body(acc, cnt, x, Wl, Wr, sb, bb, out, inv_out):
        cntb = cnt[0] + cnt[1]
        inv = 1.0 / jnp.maximum(cntb, 1.0)
        inv_out[...] = inv
        mean = (acc[0] + acc[1]) * inv
        dn = (((1,), (1,)), ((), ()))
        t = lax.dot_general(mean, Wl[...], dn,
                            preferred_element_type=jnp.float32)
        t = t + lax.dot_general(x[...], Wr[...], dn,
                                preferred_element_type=jnp.float32)
        t = t * sb[...] + bb[...]
        out[...] = jnp.maximum(t, 0.0)

    return pl.pallas_call(
        body,
        grid=(N // R,),
        in_specs=[
            pl.BlockSpec((2, R, D), lambda i: (0, i, 0)),
            pl.BlockSpec((2, R, 1), lambda i: (0, i, 0)),
            pl.BlockSpec((R, D), lambda i: (i, 0)),
            pl.BlockSpec((H, D), lambda i: (0, 0)),
            pl.BlockSpec((H, D), lambda i: (0, 0)),
            pl.BlockSpec((1, H), lambda i: (0, 0)),
            pl.BlockSpec((1, H), lambda i: (0, 0)),
        ],
        out_specs=[
            pl.BlockSpec((R, H), lambda i: (i, 0)),
            pl.BlockSpec((R, 1), lambda i: (i, 0)),
        ],
        out_shape=[
            jax.ShapeDtypeStruct((N, H), jnp.float32),
            jax.ShapeDtypeStruct((N, 1), jnp.float32),
        ],
    )


def _make_tc_layer2(N, H, Z):
    R = _largest_div(N, 1024, 8)

    def body(acc, inv, h, Wl, Wr, bb, out):
        mean = (acc[0] + acc[1]) * inv[...]
        dn = (((1,), (1,)), ((), ()))
        t = lax.dot_general(mean, Wl[...], dn,
                            preferred_element_type=jnp.float32)
        t = t + lax.dot_general(h[...], Wr[...], dn,
                                preferred_element_type=jnp.float32)
        out[...] = t + bb[...]

    return pl.pallas_call(
        body,
        grid=(N // R,),
        in_specs=[
            pl.BlockSpec((2, R, H), lambda i: (0, i, 0)),
            pl.BlockSpec((R, 1), lambda i: (i, 0)),
            pl.BlockSpec((R, H), lambda i: (i, 0)),
            pl.BlockSpec((Z, H), lambda i: (0, 0)),
            pl.BlockSpec((Z, H), lambda i: (0, 0)),
            pl.BlockSpec((1, Z), lambda i: (0, 0)),
        ],
        out_specs=pl.BlockSpec((R, Z), lambda i: (i, 0)),
        out_shape=jax.ShapeDtypeStruct((N, Z), jnp.float32),
    )


def kernel(x, x_edge_index, W1l, b1, W1r, bn_gamma, bn_beta, bn_mean, bn_var,
           W2l, b2, W2r):
    N, D = x.shape
    E = x_edge_index.shape[1]
    H = W1l.shape[0]
    Z = W2l.shape[0]

    sc_agg1 = _make_sc_agg(N, E, D, with_counts=True)
    sc_agg2 = _make_sc_agg(N, E, H, with_counts=False)
    tc1 = _make_tc_layer1(N, D, H)
    tc2 = _make_tc_layer2(N, H, Z)

    ei_flat = x_edge_index.reshape(2 * E)
    acc1, cntp = sc_agg1(x, ei_flat)
    cnt3 = cntp.reshape(NC, N, 1)

    s1 = bn_gamma / jnp.sqrt(bn_var + 1e-5)
    sb1 = s1.reshape(1, H)
    bb1 = ((b1 - bn_mean) * s1 + bn_beta).reshape(1, H)

    h, inv = tc1(acc1, cnt3, x, W1l, W1r, sb1, bb1)

    (acc2,) = sc_agg2(h, ei_flat)
    z = tc2(acc2, inv, h, W2l, W2r, b2.reshape(1, Z))
    return z
